# baseline (speedup 1.0000x reference)
import functools

import jax
import jax.numpy as jnp
from jax import lax
from jax.experimental import pallas as pl
from jax.experimental.pallas import tpu as pltpu
from jax.experimental.pallas import tpu_sc as plsc

VOCAB = 256
L = 2097152

_NC = 2
_NS = 16
_NW = _NC * _NS
_CH = L // _NW
_GB = 128
_INNER = 16
_OUTER = _CH // (_GB * _INNER)
_STAGE = _GB * _INNER
_HALO = 16


def _body(tbl_hbm, x_hbm, out_hbm, xv, idx_v, gstage, sem):
    wid = lax.axis_index("s") * _NC + lax.axis_index("c")
    base = wid * _CH

    pltpu.sync_copy(x_hbm.at[pl.ds(base, _CH)], xv.at[pl.ds(_HALO, _CH)])

    @pl.when(wid > 0)
    def _():
        pltpu.sync_copy(x_hbm.at[pl.ds(base - _HALO, _HALO)],
                        xv.at[pl.ds(0, _HALO)])

    @pl.when(wid == 0)
    def _():
        v0 = xv[pl.ds(_HALO, 16)]
        xv[pl.ds(0, 16)] = jnp.full((16,), 0, jnp.int32) + v0[0]

    def outer(ob, _):
        def inner(ib, _):
            off = (ob * _INNER + ib) * _GB
            for g in range(_GB // 16):
                o = off + g * 16
                a = xv[pl.ds(_HALO - 2 + o, 16)]
                b = xv[pl.ds(_HALO - 1 + o, 16)]
                c = xv[pl.ds(_HALO + o, 16)]
                idx_v[pl.ds(g * 16, 16)] = a * 65536 + b * 256 + c
            pltpu.async_copy(tbl_hbm.at[idx_v],
                             gstage.at[pl.ds(ib * _GB, _GB)], sem).wait()
            return 0
        lax.fori_loop(0, _INNER, inner, 0)
        pltpu.sync_copy(gstage, out_hbm.at[pl.ds(base + ob * _STAGE, _STAGE)])
        return 0

    lax.fori_loop(0, _OUTER, outer, 0)


@jax.jit
def _ngram_lookup(tbl, x):
    mesh = plsc.VectorSubcoreMesh(core_axis_name="c", subcore_axis_name="s")
    return pl.kernel(
        _body,
        out_type=jax.ShapeDtypeStruct((L,), jnp.float32),
        mesh=mesh,
        scratch_types=[
            pltpu.VMEM((_HALO + _CH,), jnp.int32),
            pltpu.VMEM((_GB,), jnp.int32),
            pltpu.VMEM((_STAGE,), jnp.float32),
            pltpu.SemaphoreType.DMA,
        ],
    )(tbl, x)


def kernel(probs, x):
    return _ngram_lookup(probs.reshape(-1), x)


# speedup vs baseline: 1.6239x; 1.6239x over previous
import functools

import jax
import jax.numpy as jnp
from jax import lax
from jax.experimental import pallas as pl
from jax.experimental.pallas import tpu as pltpu
from jax.experimental.pallas import tpu_sc as plsc

VOCAB = 256
L = 2097152

_NC = 2
_NS = 16
_NW = _NC * _NS
_CH = L // _NW
_GB = 128
_INNER = 16
_OUTER = _CH // (_GB * _INNER)
_STAGE = _GB * _INNER
_HALO = 16


def _body(tbl_hbm, x_hbm, out_hbm, xv, idx_v, gstage, sem0, sem1):
    sem = (sem0, sem1)
    wid = lax.axis_index("s") * _NC + lax.axis_index("c")
    base = wid * _CH

    pltpu.sync_copy(x_hbm.at[pl.ds(base, _CH)], xv.at[pl.ds(_HALO, _CH)])

    @pl.when(wid > 0)
    def _():
        pltpu.sync_copy(x_hbm.at[pl.ds(base - _HALO, _HALO)],
                        xv.at[pl.ds(0, _HALO)])

    @pl.when(wid == 0)
    def _():
        v0 = xv[pl.ds(_HALO, 16)]
        xv[pl.ds(0, 16)] = jnp.full((16,), 0, jnp.int32) + v0[0]

    def gather(j):
        return pltpu.make_async_copy(tbl_hbm.at[idx_v.at[j]],
                                     gstage.at[pl.ds(j * _GB, _GB)],
                                     sem[j % 2])

    def outer(ob, _):
        def grpfn(g, _):
            o = ob * _STAGE + g * 16
            a = xv[pl.ds(_HALO - 2 + o, 16)]
            b = xv[pl.ds(_HALO - 1 + o, 16)]
            c = xv[pl.ds(_HALO + o, 16)]
            idx_v[g // 8, pl.ds((g % 8) * 16, 16)] = a * 65536 + b * 256 + c
            return 0
        lax.fori_loop(0, _STAGE // 16, grpfn, 0)
        gather(0).start()
        for j in range(1, _INNER):
            gather(j).start()
            gather(j - 1).wait()
        gather(_INNER - 1).wait()
        pltpu.sync_copy(gstage, out_hbm.at[pl.ds(base + ob * _STAGE, _STAGE)])
        return 0

    lax.fori_loop(0, _OUTER, outer, 0)


@jax.jit
def _ngram_lookup(tbl, x):
    mesh = plsc.VectorSubcoreMesh(core_axis_name="c", subcore_axis_name="s")
    return pl.kernel(
        _body,
        out_type=jax.ShapeDtypeStruct((L,), jnp.float32),
        mesh=mesh,
        scratch_types=[
            pltpu.VMEM((_HALO + _CH,), jnp.int32),
            pltpu.VMEM((_INNER, _GB), jnp.int32),
            pltpu.VMEM((_STAGE,), jnp.float32),
            pltpu.SemaphoreType.DMA,
            pltpu.SemaphoreType.DMA,
        ],
    )(tbl, x)


def kernel(probs, x):
    return _ngram_lookup(probs.reshape(-1), x)


# speedup vs baseline: 2.2129x; 1.3627x over previous
import functools

import jax
import jax.numpy as jnp
from jax import lax
from jax.experimental import pallas as pl
from jax.experimental.pallas import tpu as pltpu
from jax.experimental.pallas import tpu_sc as plsc

VOCAB = 256
L = 2097152

_NC = 2
_NS = 16
_NW = _NC * _NS
_CH = L // _NW
_GB = 128
_INNER = 16
_OUTER = _CH // (_GB * _INNER)
_STAGE = _GB * _INNER
_HALO = 16


def _body(tbl_hbm, x_hbm, out_hbm, xv, idx_v, gstage, sem0, sem1, sem2,
          sem3):
    sem = (sem0, sem1, sem2, sem3)
    wid = lax.axis_index("s") * _NC + lax.axis_index("c")
    base = wid * _CH

    pltpu.sync_copy(x_hbm.at[pl.ds(base, _CH)], xv.at[pl.ds(_HALO, _CH)])

    @pl.when(wid > 0)
    def _():
        pltpu.sync_copy(x_hbm.at[pl.ds(base - _HALO, _HALO)],
                        xv.at[pl.ds(0, _HALO)])

    @pl.when(wid == 0)
    def _():
        v0 = xv[pl.ds(_HALO, 16)]
        xv[pl.ds(0, 16)] = jnp.full((16,), 0, jnp.int32) + v0[0]

    def gather(j):
        return pltpu.make_async_copy(tbl_hbm.at[idx_v.at[j]],
                                     gstage.at[pl.ds(j * _GB, _GB)],
                                     sem[j % 4])

    def outer(ob, _):
        def grpfn(g, _):
            o = ob * _STAGE + g * 16
            a = xv[pl.ds(_HALO - 2 + o, 16)]
            b = xv[pl.ds(_HALO - 1 + o, 16)]
            c = xv[pl.ds(_HALO + o, 16)]
            idx_v[g // 8, pl.ds((g % 8) * 16, 16)] = a * 65536 + b * 256 + c
            return 0
        lax.fori_loop(0, _STAGE // 16, grpfn, 0)
        for j in range(3):
            gather(j).start()
        for j in range(3, _INNER):
            gather(j).start()
            gather(j - 3).wait()
        for j in range(_INNER - 3, _INNER):
            gather(j).wait()
        pltpu.sync_copy(gstage, out_hbm.at[pl.ds(base + ob * _STAGE, _STAGE)])
        return 0

    lax.fori_loop(0, _OUTER, outer, 0)


@jax.jit
def _ngram_lookup(tbl, x):
    mesh = plsc.VectorSubcoreMesh(core_axis_name="c", subcore_axis_name="s")
    return pl.kernel(
        _body,
        out_type=jax.ShapeDtypeStruct((L,), jnp.float32),
        mesh=mesh,
        scratch_types=[
            pltpu.VMEM((_HALO + _CH,), jnp.int32),
            pltpu.VMEM((_INNER, _GB), jnp.int32),
            pltpu.VMEM((_STAGE,), jnp.float32),
            pltpu.SemaphoreType.DMA,
            pltpu.SemaphoreType.DMA,
            pltpu.SemaphoreType.DMA,
            pltpu.SemaphoreType.DMA,
        ],
    )(tbl, x)


def kernel(probs, x):
    return _ngram_lookup(probs.reshape(-1), x)


# speedup vs baseline: 2.6420x; 1.1939x over previous
import functools

import jax
import jax.numpy as jnp
from jax import lax
from jax.experimental import pallas as pl
from jax.experimental.pallas import tpu as pltpu
from jax.experimental.pallas import tpu_sc as plsc

VOCAB = 256
L = 2097152

_NC = 2
_NS = 16
_NW = _NC * _NS
_CH = L // _NW
_GB = 128
_INNER = 16
_OUTER = _CH // (_GB * _INNER)
_STAGE = _GB * _INNER
_HALO = 16


def _body(tbl_hbm, x_hbm, out_hbm, xv, idx_v, gstage, *sems):
    sem = sems
    wid = lax.axis_index("s") * _NC + lax.axis_index("c")
    base = wid * _CH

    pltpu.sync_copy(x_hbm.at[pl.ds(base, _CH)], xv.at[pl.ds(_HALO, _CH)])

    @pl.when(wid > 0)
    def _():
        pltpu.sync_copy(x_hbm.at[pl.ds(base - _HALO, _HALO)],
                        xv.at[pl.ds(0, _HALO)])

    @pl.when(wid == 0)
    def _():
        v0 = xv[pl.ds(_HALO, 16)]
        xv[pl.ds(0, 16)] = jnp.full((16,), 0, jnp.int32) + v0[0]

    def gather(j):
        return pltpu.make_async_copy(tbl_hbm.at[idx_v.at[j]],
                                     gstage.at[pl.ds(j * _GB, _GB)],
                                     sem[j % 8])

    def outer(ob, _):
        def grpfn(g, _):
            o = ob * _STAGE + g * 16
            a = xv[pl.ds(_HALO - 2 + o, 16)]
            b = xv[pl.ds(_HALO - 1 + o, 16)]
            c = xv[pl.ds(_HALO + o, 16)]
            idx_v[g // 8, pl.ds((g % 8) * 16, 16)] = a * 65536 + b * 256 + c
            return 0
        lax.fori_loop(0, _STAGE // 16, grpfn, 0)
        for j in range(7):
            gather(j).start()
        for j in range(7, _INNER):
            gather(j).start()
            gather(j - 7).wait()
        for j in range(_INNER - 7, _INNER):
            gather(j).wait()
        pltpu.sync_copy(gstage, out_hbm.at[pl.ds(base + ob * _STAGE, _STAGE)])
        return 0

    lax.fori_loop(0, _OUTER, outer, 0)


@jax.jit
def _ngram_lookup(tbl, x):
    mesh = plsc.VectorSubcoreMesh(core_axis_name="c", subcore_axis_name="s")
    return pl.kernel(
        _body,
        out_type=jax.ShapeDtypeStruct((L,), jnp.float32),
        mesh=mesh,
        scratch_types=[
            pltpu.VMEM((_HALO + _CH,), jnp.int32),
            pltpu.VMEM((_INNER, _GB), jnp.int32),
            pltpu.VMEM((_STAGE,), jnp.float32),
        ] + [pltpu.SemaphoreType.DMA] * 8,
    )(tbl, x)


def kernel(probs, x):
    return _ngram_lookup(probs.reshape(-1), x)
